# baseline (speedup 1.0000x reference)
import functools

import jax
import jax.numpy as jnp
import numpy as np
from jax import lax
from jax.experimental import pallas as pl
from jax.experimental.pallas import tpu as pltpu
from jax.experimental.pallas import tpu_sc as plsc

VOCAB = 100000
D0, D1, D2 = 64, 96, 128
AGG = 128
BATCH = 4096

_R0 = VOCAB * D0 // AGG
_R1 = VOCAB * D1 // AGG

_info = plsc.get_sparse_core_info()
_NC, _NS, _L = _info.num_cores, _info.num_subcores, _info.num_lanes
_NW = _NC * _NS
_BPW = BATCH // _NW

_THIRD = float(np.float32(1.0) / np.float32(3.0))


def _mean_kernel(idx_hbm, w0_hbm, w1_hbm, w2_hbm, out_hbm,
                 idx_v, q0_v, q1a_v, q1b_v, r0, r1a, r1b, r2, sem):
    wid = lax.axis_index("s") * _NC + lax.axis_index("c")
    base = wid * _BPW

    pltpu.sync_copy(idx_hbm.at[pl.ds(base, _BPW)], idx_v)
    for k in range(_BPW // _L):
        sl = pl.ds(k * _L, _L)
        vec = idx_v[sl]
        q0_v[sl] = lax.shift_right_logical(vec, 1)
        t = lax.shift_right_logical(vec * 3, 2)
        q1a_v[sl] = t
        q1b_v[sl] = lax.min(t + 1, jnp.full((_L,), _R1 - 1, jnp.int32))

    c0 = pltpu.async_copy(w0_hbm.at[q0_v], r0, sem)
    c1a = pltpu.async_copy(w1_hbm.at[q1a_v], r1a, sem)
    c1b = pltpu.async_copy(w1_hbm.at[q1b_v], r1b, sem)
    c2 = pltpu.async_copy(w2_hbm.at[idx_v], r2, sem)
    c0.wait()
    c1a.wait()
    c1b.wait()
    c2.wait()

    third = jnp.float32(_THIRD)

    def row(r, carry):
        rb = pl.multiple_of((r // _L) * _L, _L)
        vec = idx_v[pl.ds(rb, _L)]
        msk = lax.iota(jnp.int32, _L) == lax.rem(r, _L)
        i = jnp.sum(jnp.where(msk, vec, 0))
        o0 = lax.mul(lax.bitwise_and(i, 1), D0)
        o1 = lax.rem(lax.mul(i, D1), D2)
        for j in range(AGG // _L):
            c = j * _L
            v = r2[r, pl.ds(c, _L)]
            if c < D1:
                g = o1 + c
                ga = pl.multiple_of(lax.min(g, D2 - _L), _L)
                gb = pl.multiple_of(lax.max(g - D2, 0), _L)
                va = r1a[r, pl.ds(ga, _L)]
                vb = r1b[r, pl.ds(gb, _L)]
                v = v + jnp.where(g < D2, va, vb)
            if c < D0:
                g0 = pl.multiple_of(o0 + c, _L)
                v = v + r0[r, pl.ds(g0, _L)]
            r2[r, pl.ds(c, _L)] = v * third
        return carry

    lax.fori_loop(0, _BPW, row, 0, unroll=2)

    pltpu.sync_copy(r2, out_hbm.at[pl.ds(base, _BPW)])


@jax.jit
def kernel(indexes, W0, W1, W2):
    idx = indexes.astype(jnp.int32)
    w0 = W0.reshape(_R0, AGG)
    w1 = W1.reshape(_R1, AGG)
    mesh = plsc.VectorSubcoreMesh(core_axis_name="c", subcore_axis_name="s")
    f = functools.partial(
        pl.kernel,
        mesh=mesh,
        out_type=jax.ShapeDtypeStruct((BATCH, AGG), jnp.float32),
        scratch_types=[
            pltpu.VMEM((_BPW,), jnp.int32),
            pltpu.VMEM((_BPW,), jnp.int32),
            pltpu.VMEM((_BPW,), jnp.int32),
            pltpu.VMEM((_BPW,), jnp.int32),
            pltpu.VMEM((_BPW, AGG), jnp.float32),
            pltpu.VMEM((_BPW, AGG), jnp.float32),
            pltpu.VMEM((_BPW, AGG), jnp.float32),
            pltpu.VMEM((_BPW, AGG), jnp.float32),
            pltpu.SemaphoreType.DMA,
        ],
        compiler_params=pltpu.CompilerParams(needs_layout_passes=False),
    )(_mean_kernel)
    return f(idx, w0, w1, W2)
